# baseline (device time: 14387 ns/iter reference)
import jax
import jax.numpy as jnp
from jax import lax
from jax.experimental import pallas as pl
from jax.experimental.pallas import tpu as pltpu

NC = 8


def kernel(A, B):
    m, k = A.shape
    k2, n = B.shape
    rw = m // NC

    def body(a_hbm, b_hbm, out_hbm,
             a_v, b_v, out_v,
             send1, recv1, send2, recv2,
             in_sems, out_sems,
             s1s, s1r, s2s, s2r):
        my_pos = lax.axis_index("i")
        px = my_pos ^ 1
        py = 3 - my_pos

        cp_a = pltpu.make_async_copy(a_hbm, a_v, in_sems.at[0])
        cp_a.start()
        cp_b = pltpu.make_async_copy(b_hbm, b_v, in_sems.at[1])
        cp_b.start()

        barrier = pltpu.get_barrier_semaphore()
        for nbr in (px, py):
            pl.semaphore_signal(
                barrier, inc=1,
                device_id=nbr, device_id_type=pl.DeviceIdType.LOGICAL,
            )
        pl.semaphore_wait(barrier, 2)

        cp_a.wait()
        cp_b.wait()
        b = b_v[...].astype(jnp.bfloat16)

        order = []
        for i in range(NC // 2):
            order.append((2 * i, px, py))
            order.append((2 * i + 1, py, px))

        rdma1 = {}
        for c, first, _second in order:
            ac = a_v[c * rw:(c + 1) * rw, :].astype(jnp.bfloat16)
            pc = jnp.dot(ac, b, preferred_element_type=jnp.float32)
            send1[c] = pc.astype(jnp.bfloat16)
            r = pltpu.make_async_remote_copy(
                src_ref=send1.at[c],
                dst_ref=recv1.at[c],
                send_sem=s1s.at[c],
                recv_sem=s1r.at[c],
                device_id=first,
                device_id_type=pl.DeviceIdType.LOGICAL,
            )
            r.start()
            rdma1[c] = r

        rdma2 = {}
        for c, _first, second in order:
            rdma1[c].wait_recv()
            send2[c] = send1[c] + recv1[c]
            r = pltpu.make_async_remote_copy(
                src_ref=send2.at[c],
                dst_ref=recv2.at[c],
                send_sem=s2s.at[c],
                recv_sem=s2r.at[c],
                device_id=second,
                device_id_type=pl.DeviceIdType.LOGICAL,
            )
            r.start()
            rdma2[c] = r

        out_cps = []
        for c, _first, _second in order:
            rdma2[c].wait_recv()
            total = send2[c] + recv2[c]
            out_v[c] = jnp.maximum(total, 0.0)
            cp = pltpu.make_async_copy(
                out_v.at[c],
                out_hbm.at[pl.ds(c * rw, rw), :],
                out_sems.at[c],
            )
            cp.start()
            out_cps.append(cp)

        for cp in out_cps:
            cp.wait()

        for c in range(NC):
            rdma1[c].wait_send()
            rdma2[c].wait_send()

    return pl.pallas_call(
        body,
        out_shape=jax.ShapeDtypeStruct((m, n), jnp.bfloat16),
        in_specs=[
            pl.BlockSpec(memory_space=pl.ANY),
            pl.BlockSpec(memory_space=pl.ANY),
        ],
        out_specs=pl.BlockSpec(memory_space=pl.ANY),
        scratch_shapes=[
            pltpu.VMEM((m, k), jnp.float32),
            pltpu.VMEM((k, n), jnp.float32),
            pltpu.VMEM((NC, rw, n), jnp.bfloat16),
            pltpu.VMEM((NC, rw, n), jnp.bfloat16),
            pltpu.VMEM((NC, rw, n), jnp.bfloat16),
            pltpu.VMEM((NC, rw, n), jnp.bfloat16),
            pltpu.VMEM((NC, rw, n), jnp.bfloat16),
            pltpu.SemaphoreType.DMA((2,)),
            pltpu.SemaphoreType.DMA((NC,)),
            pltpu.SemaphoreType.DMA((NC,)),
            pltpu.SemaphoreType.DMA((NC,)),
            pltpu.SemaphoreType.DMA((NC,)),
            pltpu.SemaphoreType.DMA((NC,)),
        ],
        compiler_params=pltpu.CompilerParams(collective_id=0),
    )(A, B)
